# baseline (speedup 1.0000x reference)
import jax
import jax.numpy as jnp
from jax import lax
from jax.experimental import pallas as pl
from jax.experimental.pallas import tpu as pltpu
from jax.experimental.pallas import tpu_sc as plsc

NU = 25000
NI = 25000
N = NU + NI
NP = 51200
STRIPE = NP // 16
H = 32
E = 800000
CHUNK = 192
NCH = 264
EP = 16 * NCH * CHUNK
BB = 4096
PT = BB // 16
DUMMY = N + 40

_mesh = plsc.VectorSubcoreMesh(core_axis_name="c", subcore_axis_name="s")


def _sc_body(users_r, items_r, src_r, dst_r, val_r, emb_lo, emb_hi,
             u_parts, i_parts, elay_lo, elay_hi,
             agg_s, rows0, rows1, rows2, rows3, sidx4, didx4, val4, bidx,
             gsem0, gsem1, gsem2, gsem3, ssem0, ssem1, ssem2, ssem3,
             vsem0, vsem1, vsem2, vsem3, dsem0, dsem1, dsem2, dsem3):
    c = lax.axis_index("c")
    s = lax.axis_index("s")
    rows = (rows0, rows1, rows2, rows3)
    gsem = (gsem0, gsem1, gsem2, gsem3)
    ssem = (ssem0, ssem1, ssem2, ssem3)
    vsem = (vsem0, vsem1, vsem2, vsem3)
    dsem = (dsem0, dsem1, dsem2, dsem3)
    z16 = jnp.zeros((16,), jnp.float32)
    base = s * STRIPE

    def one_half(table, elay, upart_row, ipart_row):
        def seed(k, _):
            pltpu.sync_copy(table.at[pl.ds(base + k * 192, 192)],
                            rows0)
            pltpu.sync_copy(rows0,
                            elay.at[0].at[pl.ds(base + k * 192, 192)])
            return 0

        lax.fori_loop(0, 16, seed, 0)
        pltpu.sync_copy(table.at[pl.ds(base + 3072, 128)],
                        rows0.at[pl.ds(0, 128)])
        pltpu.sync_copy(rows0.at[pl.ds(0, 128)],
                        elay.at[0].at[pl.ds(base + 3072, 128)])
        plsc.subcore_barrier()

        def _maybe(guard, fire):
            if guard is True:
                fire()
            elif guard is not False:
                pl.when(guard)(fire)

        def pidx_sv(j, k, guard=True):
            def fire():
                blk = s * NCH + k
                pltpu.async_copy(src_r.at[blk], sidx4.at[j], vsem[j])
                pltpu.async_copy(val_r.at[blk], val4.at[j], vsem[j])
            _maybe(guard, fire)

        def pidx_d(j, k, guard=True):
            def fire():
                pltpu.async_copy(dst_r.at[s * NCH + k], didx4.at[j], dsem[j])
            _maybe(guard, fire)

        def wait_sv(j, k):
            blk = s * NCH + k
            pltpu.make_async_copy(src_r.at[blk], sidx4.at[j], vsem[j]).wait()
            pltpu.make_async_copy(val_r.at[blk], val4.at[j], vsem[j]).wait()

        def wait_scatter(j):
            for jj in range(2):
                pltpu.make_async_copy(rows[j].at[pl.ds(jj * 96, 96)],
                                      agg_s.at[didx4.at[j].at[jj]],
                                      ssem[j]).wait()

        def fgather(j, k, src_view, first=False, guard=True):
            def fire():
                if not first:
                    wait_scatter(j)
                wait_sv(j, k)
                for jj in range(2):
                    pltpu.async_copy(src_view.at[sidx4.at[j].at[jj]],
                                     rows[j].at[pl.ds(jj * 96, 96)],
                                     gsem[j])
            _maybe(guard, fire)

        def consume(j, k, src_view):
            for jj in range(2):
                pltpu.make_async_copy(src_view.at[sidx4.at[j].at[jj]],
                                      rows[j].at[pl.ds(jj * 96, 96)],
                                      gsem[j]).wait()
            pltpu.make_async_copy(dst_r.at[s * NCH + k], didx4.at[j],
                                  dsem[j]).wait()

            def scale16(i, _):
                vv = val4[j, pl.ds(i * 16, 16)]
                for e16 in range(16):
                    e = i * 16 + e16
                    v = vv[e16]
                    rows[j][e, pl.ds(0, 16)] = rows[j][e, pl.ds(0, 16)] * v
                    rows[j][e, pl.ds(16, 16)] = rows[j][e, pl.ds(16, 16)] * v
                return 0

            lax.fori_loop(0, CHUNK // 16, scale16, 0, unroll=3)
            for jj in range(2):
                pltpu.async_copy(rows[j].at[pl.ds(jj * 96, 96)],
                                 agg_s.at[didx4.at[j].at[jj]], ssem[j],
                                 add=True)

        def estep(i4, j, src_view):
            k = i4 + j
            consume(j, k, src_view)
            if isinstance(k, int):
                gsv = k + 4 < NCH
                gf = k + 3 < NCH
            else:
                gsv = k + 4 < NCH
                gf = k + 3 < NCH
            pidx_sv(j, k + 4, guard=gsv)
            j3 = (j + 3) % 4
            fgather(j3, k + 3, src_view, guard=gf)
            pidx_d(j3, k + 3, guard=gf)

        def layer(l, _):
            src_view = elay.at[l]
            dst_view = elay.at[l + 1]

            def zfill(r, _):
                rows0[r, pl.ds(0, 16)] = z16
                rows0[r, pl.ds(16, 16)] = z16
                return 0

            lax.fori_loop(0, CHUNK, zfill, 0, unroll=8)

            def zgroup(g, _):
                for jz in range(4):
                    pltpu.async_copy(
                        rows0, agg_s.at[pl.ds(base + (g * 4 + jz) * 192,
                                              192)], gsem0)
                for jz in range(4):
                    pltpu.make_async_copy(
                        rows0, agg_s.at[pl.ds(base + (g * 4 + jz) * 192,
                                              192)], gsem0).wait()
                return 0

            lax.fori_loop(0, 4, zgroup, 0)
            pltpu.sync_copy(rows0.at[pl.ds(0, 128)],
                            agg_s.at[pl.ds(base + 3072, 128)])
            plsc.subcore_barrier()

            for j in range(4):
                pidx_sv(j, j, guard=True)
            for j in range(3):
                pidx_d(j, j, guard=True)
            for j in range(3):
                fgather(j, j, src_view, first=True)

            consume(0, 0, src_view)
            pidx_sv(0, 4, guard=True)
            fgather(3, 3, src_view, first=True)
            pidx_d(3, 3, guard=True)
            for j in (1, 2, 3):
                estep(0, j, src_view)

            def pipe(i, _):
                i4 = i * 4
                for j in range(4):
                    estep(i4, j, src_view)
                return 0

            lax.fori_loop(1, NCH // 4, pipe, 0)
            for j in range(4):
                wait_scatter(j)
            plsc.subcore_barrier()

            def blend_chunk(rb, sz):
                pltpu.sync_copy(src_view.at[pl.ds(rb, sz)],
                                rows0.at[pl.ds(0, sz)])
                pltpu.sync_copy(agg_s.at[pl.ds(rb, sz)],
                                rows1.at[pl.ds(0, sz)])

                def blend_row(r, _):
                    for hh in (0, 16):
                        p = rows0[r, pl.ds(hh, 16)]
                        a = rows1[r, pl.ds(hh, 16)]
                        rows0[r, pl.ds(hh, 16)] = 0.2 * p + 0.8 * a
                    return 0

                lax.fori_loop(0, sz, blend_row, 0, unroll=4)
                pltpu.sync_copy(rows0.at[pl.ds(0, sz)],
                                dst_view.at[pl.ds(rb, sz)])

            def blend_k(k, _):
                blend_chunk(base + k * 192, 192)
                return 0

            lax.fori_loop(0, 16, blend_k, 0)
            blend_chunk(base + 3072, 128)
            plsc.subcore_barrier()
            return 0

        lax.fori_loop(0, 3, layer, 0)

        for hb in range(2):
            pltpu.sync_copy(users_r.at[pl.ds(s * 2 + hb, 1)],
                            bidx.at[pl.ds(0, 1)])
            pltpu.sync_copy(items_r.at[pl.ds(s * 2 + hb, 1)],
                            bidx.at[pl.ds(1, 1)])

            def gather4(idx_row, part_row):
                for t in range(4):
                    pltpu.async_copy(elay.at[t].at[idx_row],
                                     rows[t].at[pl.ds(0, 128)], gsem[t])
                for t in range(4):
                    pltpu.make_async_copy(elay.at[t].at[idx_row],
                                          rows[t].at[pl.ds(0, 128)],
                                          gsem[t]).wait()

                def sum_row(r, _):
                    for hh in (0, 16):
                        a = rows0[r, pl.ds(hh, 16)]
                        a = a + rows1[r, pl.ds(hh, 16)]
                        a = a + rows2[r, pl.ds(hh, 16)]
                        a = a + rows3[r, pl.ds(hh, 16)]
                        rows0[r, pl.ds(hh, 16)] = a
                    return 0

                lax.fori_loop(0, 128, sum_row, 0, unroll=2)
                off = s * PT + hb * 128
                pltpu.sync_copy(rows0.at[pl.ds(0, 128)],
                                part_row.at[pl.ds(off, 128)])

            gather4(bidx.at[0], upart_row)
            gather4(bidx.at[1], ipart_row)

    @pl.when(c == 0)
    def _():
        one_half(emb_lo, elay_lo, u_parts.at[0], i_parts.at[0])

    @pl.when(c == 1)
    def _():
        one_half(emb_hi, elay_hi, u_parts.at[1], i_parts.at[1])


@jax.jit
def kernel(users, items, edge_index, edge_vals, user_emb, item_emb):
    all_emb = jnp.concatenate([user_emb, item_emb], axis=0)
    all_emb = jnp.pad(all_emb, ((0, NP - N), (0, 0)))
    emb_lo = all_emb[:, :H]
    emb_hi = all_emb[:, H:]

    dst = edge_index[0].astype(jnp.int32)
    src = edge_index[1].astype(jnp.int32)
    pad = EP - E
    src_r = jnp.pad(src, (0, pad)).reshape(16 * NCH, 2, 96)
    dst_r = jnp.pad(dst, (0, pad), constant_values=DUMMY).reshape(
        16 * NCH, 2, 96)
    val_r = jnp.pad(edge_vals, (0, pad)).reshape(16 * NCH, CHUNK)

    users_r = users.astype(jnp.int32).reshape(32, 128)
    items_r = (items.astype(jnp.int32) + NU).reshape(32, 128)

    lf32 = jax.ShapeDtypeStruct((4, NP, H), jnp.float32)
    bf32 = jax.ShapeDtypeStruct((2, BB, H), jnp.float32)
    out_type = (bf32, bf32, lf32, lf32)

    run = pl.kernel(
        _sc_body,
        out_type=out_type,
        mesh=_mesh,
        scratch_types=(
            pltpu.VMEM_SHARED((NP, H), jnp.float32),
            pltpu.VMEM((CHUNK, H), jnp.float32),
            pltpu.VMEM((CHUNK, H), jnp.float32),
            pltpu.VMEM((CHUNK, H), jnp.float32),
            pltpu.VMEM((CHUNK, H), jnp.float32),
            pltpu.VMEM((4, 2, 96), jnp.int32),
            pltpu.VMEM((4, 2, 96), jnp.int32),
            pltpu.VMEM((4, CHUNK), jnp.float32),
            pltpu.VMEM((2, 128), jnp.int32),
        ) + (pltpu.SemaphoreType.DMA,) * 16,
        compiler_params=pltpu.CompilerParams(use_tc_tiling_on_sc=False),
        name="eghg_sc",
    )
    u_parts, i_parts, *_ = run(users_r, items_r, src_r, dst_r, val_r,
                               emb_lo, emb_hi)
    ug = jnp.concatenate([u_parts[0], u_parts[1]], axis=1)
    ig = jnp.concatenate([i_parts[0], i_parts[1]], axis=1)

    def _dot_body(u_ref, i_ref, o_ref):
        o_ref[...] = jnp.sum(u_ref[...] * i_ref[...], axis=1) * 0.0625

    gamma = pl.pallas_call(
        _dot_body,
        out_shape=jax.ShapeDtypeStruct((BB,), jnp.float32),
    )(ug, ig)
    return gamma


# speedup vs baseline: 1.5482x; 1.5482x over previous
import jax
import jax.numpy as jnp
from jax import lax
from jax.experimental import pallas as pl
from jax.experimental.pallas import tpu as pltpu
from jax.experimental.pallas import tpu_sc as plsc

NU = 25000
NI = 25000
N = NU + NI
NP = 51200
STRIPE = NP // 16
H = 32
E = 800000
CHUNK = 192
NCH = 264
EP = 16 * NCH * CHUNK
BB = 4096
PT = BB // 16
DUMMY = N + 40

_mesh = plsc.VectorSubcoreMesh(core_axis_name="c", subcore_axis_name="s")


def _sc_body(users_r, items_r, src_r, dst_r, val_r, emb_lo, emb_hi,
             u_parts, i_parts, elay_lo, elay_hi,
             agg_s, rows0, rows1, rows2, rows3, sidx4, didx4, val4, bidx,
             gsem0, gsem1, gsem2, gsem3, ssem0, ssem1, ssem2, ssem3,
             vsem0, vsem1, vsem2, vsem3, dsem0, dsem1, dsem2, dsem3):
    c = lax.axis_index("c")
    s = lax.axis_index("s")
    rows = (rows0, rows1, rows2, rows3)
    gsem = (gsem0, gsem1, gsem2, gsem3)
    ssem = (ssem0, ssem1, ssem2, ssem3)
    vsem = (vsem0, vsem1, vsem2, vsem3)
    dsem = (dsem0, dsem1, dsem2, dsem3)
    z16 = jnp.zeros((16,), jnp.float32)
    base = s * STRIPE

    def one_half(table, elay, upart_row, ipart_row):
        def seed(k, _):
            pltpu.sync_copy(table.at[pl.ds(base + k * 192, 192)],
                            rows0)
            pltpu.sync_copy(rows0,
                            elay.at[0].at[pl.ds(base + k * 192, 192)])
            return 0

        lax.fori_loop(0, 16, seed, 0)
        pltpu.sync_copy(table.at[pl.ds(base + 3072, 128)],
                        rows0.at[pl.ds(0, 128)])
        pltpu.sync_copy(rows0.at[pl.ds(0, 128)],
                        elay.at[0].at[pl.ds(base + 3072, 128)])
        plsc.subcore_barrier()

        def _maybe(guard, fire):
            if guard is True:
                fire()
            elif guard is not False:
                pl.when(guard)(fire)

        def pidx_sv(j, k, guard=True):
            def fire():
                blk = s * NCH + k
                pltpu.async_copy(src_r.at[blk], sidx4.at[j], vsem[j])
                pltpu.async_copy(val_r.at[blk], val4.at[j], vsem[j])
            _maybe(guard, fire)

        def pidx_d(j, k, guard=True):
            def fire():
                pltpu.async_copy(dst_r.at[s * NCH + k], didx4.at[j], dsem[j])
            _maybe(guard, fire)

        def wait_sv(j, k):
            blk = s * NCH + k
            pltpu.make_async_copy(src_r.at[blk], sidx4.at[j], vsem[j]).wait()
            pltpu.make_async_copy(val_r.at[blk], val4.at[j], vsem[j]).wait()

        def wait_scatter(j):
            for jj in range(2):
                pltpu.make_async_copy(rows[j].at[pl.ds(jj * 96, 96)],
                                      agg_s.at[didx4.at[j].at[jj]],
                                      ssem[j]).wait()

        def fgather(j, k, src_view, first=False, guard=True):
            def fire():
                if not first:
                    wait_scatter(j)
                wait_sv(j, k)
                for jj in range(2):
                    pltpu.async_copy(src_view.at[sidx4.at[j].at[jj]],
                                     rows[j].at[pl.ds(jj * 96, 96)],
                                     gsem[j])
            _maybe(guard, fire)

        def consume(j, k, src_view):
            for jj in range(2):
                pltpu.make_async_copy(src_view.at[sidx4.at[j].at[jj]],
                                      rows[j].at[pl.ds(jj * 96, 96)],
                                      gsem[j]).wait()
            pltpu.make_async_copy(dst_r.at[s * NCH + k], didx4.at[j],
                                  dsem[j]).wait()

            def scale16(i, _):
                vv = val4[j, pl.ds(i * 16, 16)]
                for e16 in range(16):
                    e = i * 16 + e16
                    v = vv[e16]
                    rows[j][e, pl.ds(0, 16)] = rows[j][e, pl.ds(0, 16)] * v
                    rows[j][e, pl.ds(16, 16)] = rows[j][e, pl.ds(16, 16)] * v
                return 0

            lax.fori_loop(0, CHUNK // 16, scale16, 0)
            for jj in range(2):
                pltpu.async_copy(rows[j].at[pl.ds(jj * 96, 96)],
                                 agg_s.at[didx4.at[j].at[jj]], ssem[j],
                                 add=True)

        def estep(i4, j, src_view):
            k = i4 + j
            consume(j, k, src_view)
            if isinstance(k, int):
                gsv = k + 4 < NCH
                gf = k + 3 < NCH
            else:
                gsv = k + 4 < NCH
                gf = k + 3 < NCH
            pidx_sv(j, k + 4, guard=gsv)
            j3 = (j + 3) % 4
            fgather(j3, k + 3, src_view, guard=gf)
            pidx_d(j3, k + 3, guard=gf)

        def layer(l, _):
            src_view = elay.at[l]
            dst_view = elay.at[l + 1]

            def zfill(r, _):
                rows0[r, pl.ds(0, 16)] = z16
                rows0[r, pl.ds(16, 16)] = z16
                return 0

            lax.fori_loop(0, CHUNK, zfill, 0, unroll=8)

            def zcopy(k, _):
                pltpu.sync_copy(rows0,
                                agg_s.at[pl.ds(base + k * 192, 192)])
                return 0

            lax.fori_loop(0, 16, zcopy, 0)
            pltpu.sync_copy(rows0.at[pl.ds(0, 128)],
                            agg_s.at[pl.ds(base + 3072, 128)])
            plsc.subcore_barrier()

            for j in range(4):
                pidx_sv(j, j, guard=True)
            for j in range(3):
                pidx_d(j, j, guard=True)
            for j in range(3):
                fgather(j, j, src_view, first=True)

            consume(0, 0, src_view)
            pidx_sv(0, 4, guard=True)
            fgather(3, 3, src_view, first=True)
            pidx_d(3, 3, guard=True)
            for j in (1, 2, 3):
                estep(0, j, src_view)

            def pipe(i, _):
                i4 = i * 4
                for j in range(4):
                    estep(i4, j, src_view)
                return 0

            lax.fori_loop(1, NCH // 4, pipe, 0)
            for j in range(4):
                wait_scatter(j)
            plsc.subcore_barrier()

            def blend_chunk(rb, sz):
                pltpu.sync_copy(src_view.at[pl.ds(rb, sz)],
                                rows0.at[pl.ds(0, sz)])
                pltpu.sync_copy(agg_s.at[pl.ds(rb, sz)],
                                rows1.at[pl.ds(0, sz)])

                def blend_row(r, _):
                    for hh in (0, 16):
                        p = rows0[r, pl.ds(hh, 16)]
                        a = rows1[r, pl.ds(hh, 16)]
                        rows0[r, pl.ds(hh, 16)] = 0.2 * p + 0.8 * a
                    return 0

                lax.fori_loop(0, sz, blend_row, 0, unroll=4)
                pltpu.sync_copy(rows0.at[pl.ds(0, sz)],
                                dst_view.at[pl.ds(rb, sz)])

            def blend_k(k, _):
                blend_chunk(base + k * 192, 192)
                return 0

            lax.fori_loop(0, 16, blend_k, 0)
            blend_chunk(base + 3072, 128)
            plsc.subcore_barrier()
            return 0

        lax.fori_loop(0, 3, layer, 0)

        for hb in range(2):
            pltpu.sync_copy(users_r.at[pl.ds(s * 2 + hb, 1)],
                            bidx.at[pl.ds(0, 1)])
            pltpu.sync_copy(items_r.at[pl.ds(s * 2 + hb, 1)],
                            bidx.at[pl.ds(1, 1)])

            def gather4(idx_row, part_row):
                for t in range(4):
                    pltpu.async_copy(elay.at[t].at[idx_row],
                                     rows[t].at[pl.ds(0, 128)], gsem[t])
                for t in range(4):
                    pltpu.make_async_copy(elay.at[t].at[idx_row],
                                          rows[t].at[pl.ds(0, 128)],
                                          gsem[t]).wait()

                def sum_row(r, _):
                    for hh in (0, 16):
                        a = rows0[r, pl.ds(hh, 16)]
                        a = a + rows1[r, pl.ds(hh, 16)]
                        a = a + rows2[r, pl.ds(hh, 16)]
                        a = a + rows3[r, pl.ds(hh, 16)]
                        rows0[r, pl.ds(hh, 16)] = a
                    return 0

                lax.fori_loop(0, 128, sum_row, 0, unroll=2)
                off = s * PT + hb * 128
                pltpu.sync_copy(rows0.at[pl.ds(0, 128)],
                                part_row.at[pl.ds(off, 128)])

            gather4(bidx.at[0], upart_row)
            gather4(bidx.at[1], ipart_row)

    @pl.when(c == 0)
    def _():
        one_half(emb_lo, elay_lo, u_parts.at[0], i_parts.at[0])

    @pl.when(c == 1)
    def _():
        one_half(emb_hi, elay_hi, u_parts.at[1], i_parts.at[1])


@jax.jit
def kernel(users, items, edge_index, edge_vals, user_emb, item_emb):
    all_emb = jnp.concatenate([user_emb, item_emb], axis=0)
    all_emb = jnp.pad(all_emb, ((0, NP - N), (0, 0)))
    emb_lo = all_emb[:, :H]
    emb_hi = all_emb[:, H:]

    dst = edge_index[0].astype(jnp.int32)
    src = edge_index[1].astype(jnp.int32)
    pad = EP - E
    src_r = jnp.pad(src, (0, pad)).reshape(16 * NCH, 2, 96)
    dst_r = jnp.pad(dst, (0, pad), constant_values=DUMMY).reshape(
        16 * NCH, 2, 96)
    val_r = jnp.pad(edge_vals, (0, pad)).reshape(16 * NCH, CHUNK)

    users_r = users.astype(jnp.int32).reshape(32, 128)
    items_r = (items.astype(jnp.int32) + NU).reshape(32, 128)

    lf32 = jax.ShapeDtypeStruct((4, NP, H), jnp.float32)
    bf32 = jax.ShapeDtypeStruct((2, BB, H), jnp.float32)
    out_type = (bf32, bf32, lf32, lf32)

    run = pl.kernel(
        _sc_body,
        out_type=out_type,
        mesh=_mesh,
        scratch_types=(
            pltpu.VMEM_SHARED((NP, H), jnp.float32),
            pltpu.VMEM((CHUNK, H), jnp.float32),
            pltpu.VMEM((CHUNK, H), jnp.float32),
            pltpu.VMEM((CHUNK, H), jnp.float32),
            pltpu.VMEM((CHUNK, H), jnp.float32),
            pltpu.VMEM((4, 2, 96), jnp.int32),
            pltpu.VMEM((4, 2, 96), jnp.int32),
            pltpu.VMEM((4, CHUNK), jnp.float32),
            pltpu.VMEM((2, 128), jnp.int32),
        ) + (pltpu.SemaphoreType.DMA,) * 16,
        compiler_params=pltpu.CompilerParams(use_tc_tiling_on_sc=False),
        name="eghg_sc",
    )
    u_parts, i_parts, *_ = run(users_r, items_r, src_r, dst_r, val_r,
                               emb_lo, emb_hi)
    ug = jnp.concatenate([u_parts[0], u_parts[1]], axis=1)
    ig = jnp.concatenate([i_parts[0], i_parts[1]], axis=1)

    def _dot_body(u_ref, i_ref, o_ref):
        o_ref[...] = jnp.sum(u_ref[...] * i_ref[...], axis=1) * 0.0625

    gamma = pl.pallas_call(
        _dot_body,
        out_shape=jax.ShapeDtypeStruct((BB,), jnp.float32),
    )(ug, ig)
    return gamma


# speedup vs baseline: 1.6065x; 1.0376x over previous
import jax
import jax.numpy as jnp
from jax import lax
from jax.experimental import pallas as pl
from jax.experimental.pallas import tpu as pltpu
from jax.experimental.pallas import tpu_sc as plsc

NU = 25000
NI = 25000
N = NU + NI
NP = 51200
STRIPE = NP // 16
H = 32
E = 800000
CHUNK = 192
NCH = 264
EP = 16 * NCH * CHUNK
BB = 4096
PT = BB // 16
DUMMY = N + 40

_mesh = plsc.VectorSubcoreMesh(core_axis_name="c", subcore_axis_name="s")


def _sc_body(users_r, items_r, src_r, dst_r, val_r, emb_lo, emb_hi,
             u_parts, i_parts, elay_lo, elay_hi,
             agg_s, rows0, rows1, rows2, rows3, sidx4, didx4, val4, bidx,
             gsem0, gsem1, gsem2, gsem3, ssem0, ssem1, ssem2, ssem3,
             vsem0, vsem1, vsem2, vsem3, dsem0, dsem1, dsem2, dsem3):
    c = lax.axis_index("c")
    s = lax.axis_index("s")
    rows = (rows0, rows1, rows2, rows3)
    gsem = (gsem0, gsem1, gsem2, gsem3)
    ssem = (ssem0, ssem1, ssem2, ssem3)
    vsem = (vsem0, vsem1, vsem2, vsem3)
    dsem = (dsem0, dsem1, dsem2, dsem3)
    z16 = jnp.zeros((16,), jnp.float32)
    base = s * STRIPE

    def one_half(table, elay, upart_row, ipart_row):
        def seed(k, _):
            pltpu.sync_copy(table.at[pl.ds(base + k * 192, 192)],
                            rows0)
            pltpu.sync_copy(rows0,
                            elay.at[0].at[pl.ds(base + k * 192, 192)])
            return 0

        lax.fori_loop(0, 16, seed, 0)
        pltpu.sync_copy(table.at[pl.ds(base + 3072, 128)],
                        rows0.at[pl.ds(0, 128)])
        pltpu.sync_copy(rows0.at[pl.ds(0, 128)],
                        elay.at[0].at[pl.ds(base + 3072, 128)])
        plsc.subcore_barrier()

        def _maybe(guard, fire):
            if guard is True:
                fire()
            elif guard is not False:
                pl.when(guard)(fire)

        def pidx_sv(j, k, guard=True):
            def fire():
                blk = s * NCH + k
                pltpu.async_copy(src_r.at[blk], sidx4.at[j], vsem[j])
                pltpu.async_copy(val_r.at[blk], val4.at[j], vsem[j])
            _maybe(guard, fire)

        def pidx_d(j, k, guard=True):
            def fire():
                pltpu.async_copy(dst_r.at[s * NCH + k], didx4.at[j], dsem[j])
            _maybe(guard, fire)

        def wait_sv(j, k):
            blk = s * NCH + k
            pltpu.make_async_copy(src_r.at[blk], sidx4.at[j], vsem[j]).wait()
            pltpu.make_async_copy(val_r.at[blk], val4.at[j], vsem[j]).wait()

        def wait_scatter(j):
            for jj in range(2):
                pltpu.make_async_copy(rows[j].at[pl.ds(jj * 96, 96)],
                                      agg_s.at[didx4.at[j].at[jj]],
                                      ssem[j]).wait()

        def fgather(j, k, src_view, first=False, guard=True):
            def fire():
                if not first:
                    wait_scatter(j)
                wait_sv(j, k)
                for jj in range(2):
                    pltpu.async_copy(src_view.at[sidx4.at[j].at[jj]],
                                     rows[j].at[pl.ds(jj * 96, 96)],
                                     gsem[j])
            _maybe(guard, fire)

        def consume(j, k, src_view):
            for jj in range(2):
                pltpu.make_async_copy(src_view.at[sidx4.at[j].at[jj]],
                                      rows[j].at[pl.ds(jj * 96, 96)],
                                      gsem[j]).wait()
            pltpu.make_async_copy(dst_r.at[s * NCH + k], didx4.at[j],
                                  dsem[j]).wait()

            def scale16(i, _):
                vv = val4[j, pl.ds(i * 16, 16)]
                for e16 in range(16):
                    e = i * 16 + e16
                    v = vv[e16]
                    rows[j][e, pl.ds(0, 16)] = rows[j][e, pl.ds(0, 16)] * v
                    rows[j][e, pl.ds(16, 16)] = rows[j][e, pl.ds(16, 16)] * v
                return 0

            lax.fori_loop(0, CHUNK // 16, scale16, 0)
            for jj in range(2):
                pltpu.async_copy(rows[j].at[pl.ds(jj * 96, 96)],
                                 agg_s.at[didx4.at[j].at[jj]], ssem[j],
                                 add=True)

        def estep(i4, j, src_view):
            k = i4 + j
            consume(j, k, src_view)
            if isinstance(k, int):
                gsv = k + 4 < NCH
                gf = k + 3 < NCH
            else:
                gsv = k + 4 < NCH
                gf = k + 3 < NCH
            pidx_sv(j, k + 4, guard=gsv)
            j3 = (j + 3) % 4
            fgather(j3, k + 3, src_view, guard=gf)
            pidx_d(j3, k + 3, guard=gf)

        def layer(l, _):
            src_view = elay.at[l]
            dst_view = elay.at[l + 1]

            def zfill(r, _):
                rows0[r, pl.ds(0, 16)] = z16
                rows0[r, pl.ds(16, 16)] = z16
                return 0

            lax.fori_loop(0, CHUNK, zfill, 0, unroll=8)

            def zcopy(k, _):
                pltpu.sync_copy(rows0,
                                agg_s.at[pl.ds(base + k * 192, 192)])
                return 0

            lax.fori_loop(0, 16, zcopy, 0)
            pltpu.sync_copy(rows0.at[pl.ds(0, 128)],
                            agg_s.at[pl.ds(base + 3072, 128)])
            plsc.subcore_barrier()

            for j in range(4):
                pidx_sv(j, j, guard=True)
            for j in range(3):
                pidx_d(j, j, guard=True)
            for j in range(3):
                fgather(j, j, src_view, first=True)

            consume(0, 0, src_view)
            pidx_sv(0, 4, guard=True)
            fgather(3, 3, src_view, first=True)
            pidx_d(3, 3, guard=True)
            for j in (1, 2, 3):
                estep(0, j, src_view)

            def pipe(i, _):
                i4 = i * 4
                for j in range(4):
                    estep(i4, j, src_view)
                return 0

            lax.fori_loop(1, NCH // 4, pipe, 0)
            for j in range(4):
                wait_scatter(j)
            plsc.subcore_barrier()

            bset = ((rows0, rows1, gsem0), (rows2, rows3, gsem1))

            def aload(x, k, sz):
                pv, _, ls = bset[x]
                pltpu.async_copy(src_view.at[pl.ds(base + k * 192, sz)],
                                 pv.at[pl.ds(0, sz)], ls)

            def wload(x, k, sz):
                pv, ag, ls = bset[x]
                rb = base + k * 192
                pltpu.make_async_copy(src_view.at[pl.ds(rb, sz)],
                                      pv.at[pl.ds(0, sz)], ls).wait()
                pltpu.sync_copy(agg_s.at[pl.ds(rb, sz)],
                                ag.at[pl.ds(0, sz)])

            def bcompute(x, k, sz):
                pv, ag, _ = bset[x]

                def blend_row(r, _):
                    for hh in (0, 16):
                        p = pv[r, pl.ds(hh, 16)]
                        a = ag[r, pl.ds(hh, 16)]
                        pv[r, pl.ds(hh, 16)] = 0.2 * p + 0.8 * a
                    return 0

                lax.fori_loop(0, sz, blend_row, 0, unroll=4)
                pltpu.sync_copy(pv.at[pl.ds(0, sz)],
                                dst_view.at[pl.ds(base + k * 192, sz)])

            aload(0, 0, 192)

            def bpipe(i, _):
                k0 = 2 * i
                wload(0, k0, 192)
                aload(1, k0 + 1, 192)
                bcompute(0, k0, 192)
                wload(1, k0 + 1, 192)

                @pl.when(k0 + 2 < 16)
                def _():
                    aload(0, k0 + 2, 192)

                bcompute(1, k0 + 1, 192)
                return 0

            lax.fori_loop(0, 8, bpipe, 0)
            aload(0, 16, 128)
            wload(0, 16, 128)
            bcompute(0, 16, 128)
            plsc.subcore_barrier()
            return 0

        lax.fori_loop(0, 3, layer, 0)

        for hb in range(2):
            pltpu.sync_copy(users_r.at[pl.ds(s * 2 + hb, 1)],
                            bidx.at[pl.ds(0, 1)])
            pltpu.sync_copy(items_r.at[pl.ds(s * 2 + hb, 1)],
                            bidx.at[pl.ds(1, 1)])

            def gather4(idx_row, part_row):
                for t in range(4):
                    pltpu.async_copy(elay.at[t].at[idx_row],
                                     rows[t].at[pl.ds(0, 128)], gsem[t])
                for t in range(4):
                    pltpu.make_async_copy(elay.at[t].at[idx_row],
                                          rows[t].at[pl.ds(0, 128)],
                                          gsem[t]).wait()

                def sum_row(r, _):
                    for hh in (0, 16):
                        a = rows0[r, pl.ds(hh, 16)]
                        a = a + rows1[r, pl.ds(hh, 16)]
                        a = a + rows2[r, pl.ds(hh, 16)]
                        a = a + rows3[r, pl.ds(hh, 16)]
                        rows0[r, pl.ds(hh, 16)] = a
                    return 0

                lax.fori_loop(0, 128, sum_row, 0, unroll=2)
                off = s * PT + hb * 128
                pltpu.sync_copy(rows0.at[pl.ds(0, 128)],
                                part_row.at[pl.ds(off, 128)])

            gather4(bidx.at[0], upart_row)
            gather4(bidx.at[1], ipart_row)

    @pl.when(c == 0)
    def _():
        one_half(emb_lo, elay_lo, u_parts.at[0], i_parts.at[0])

    @pl.when(c == 1)
    def _():
        one_half(emb_hi, elay_hi, u_parts.at[1], i_parts.at[1])


@jax.jit
def kernel(users, items, edge_index, edge_vals, user_emb, item_emb):
    all_emb = jnp.concatenate([user_emb, item_emb], axis=0)
    all_emb = jnp.pad(all_emb, ((0, NP - N), (0, 0)))
    emb_lo = all_emb[:, :H]
    emb_hi = all_emb[:, H:]

    dst = edge_index[0].astype(jnp.int32)
    src = edge_index[1].astype(jnp.int32)
    pad = EP - E
    src_r = jnp.pad(src, (0, pad)).reshape(16 * NCH, 2, 96)
    dst_r = jnp.pad(dst, (0, pad), constant_values=DUMMY).reshape(
        16 * NCH, 2, 96)
    val_r = jnp.pad(edge_vals, (0, pad)).reshape(16 * NCH, CHUNK)

    users_r = users.astype(jnp.int32).reshape(32, 128)
    items_r = (items.astype(jnp.int32) + NU).reshape(32, 128)

    lf32 = jax.ShapeDtypeStruct((4, NP, H), jnp.float32)
    bf32 = jax.ShapeDtypeStruct((2, BB, H), jnp.float32)
    out_type = (bf32, bf32, lf32, lf32)

    run = pl.kernel(
        _sc_body,
        out_type=out_type,
        mesh=_mesh,
        scratch_types=(
            pltpu.VMEM_SHARED((NP, H), jnp.float32),
            pltpu.VMEM((CHUNK, H), jnp.float32),
            pltpu.VMEM((CHUNK, H), jnp.float32),
            pltpu.VMEM((CHUNK, H), jnp.float32),
            pltpu.VMEM((CHUNK, H), jnp.float32),
            pltpu.VMEM((4, 2, 96), jnp.int32),
            pltpu.VMEM((4, 2, 96), jnp.int32),
            pltpu.VMEM((4, CHUNK), jnp.float32),
            pltpu.VMEM((2, 128), jnp.int32),
        ) + (pltpu.SemaphoreType.DMA,) * 16,
        compiler_params=pltpu.CompilerParams(use_tc_tiling_on_sc=False),
        name="eghg_sc",
    )
    u_parts, i_parts, *_ = run(users_r, items_r, src_r, dst_r, val_r,
                               emb_lo, emb_hi)
    ug = jnp.concatenate([u_parts[0], u_parts[1]], axis=1)
    ig = jnp.concatenate([i_parts[0], i_parts[1]], axis=1)

    def _dot_body(u_ref, i_ref, o_ref):
        o_ref[...] = jnp.sum(u_ref[...] * i_ref[...], axis=1) * 0.0625

    gamma = pl.pallas_call(
        _dot_body,
        out_shape=jax.ShapeDtypeStruct((BB,), jnp.float32),
    )(ug, ig)
    return gamma
